# baseline (device time: 9098 ns/iter reference)
import jax
import jax.numpy as jnp
from jax import lax
from jax.experimental import pallas as pl
from jax.experimental.pallas import tpu as pltpu


def kernel(x, dest):
    m, n = x.shape
    dest2 = dest.reshape(1, m).astype(jnp.int32)

    def body(x_ref, d_ref, out_ref, xs_ref, xr_ref, dr_ref, sems):
        my_x = lax.axis_index("x")
        my_y = lax.axis_index("y")
        peer = (1 - my_x, my_y)

        barrier = pltpu.get_barrier_semaphore()
        pl.semaphore_signal(
            barrier, inc=1, device_id=peer, device_id_type=pl.DeviceIdType.MESH
        )
        pl.semaphore_wait(barrier, 1)

        rd = pltpu.make_async_remote_copy(
            src_ref=d_ref,
            dst_ref=dr_ref,
            send_sem=sems.at[2],
            recv_sem=sems.at[3],
            device_id=peer,
            device_id_type=pl.DeviceIdType.MESH,
        )
        rd.start()

        xs_ref[:, :] = x_ref[:, :].astype(jnp.bfloat16)
        rx = pltpu.make_async_remote_copy(
            src_ref=xs_ref,
            dst_ref=xr_ref,
            send_sem=sems.at[0],
            recv_sem=sems.at[1],
            device_id=peer,
            device_id_type=pl.DeviceIdType.MESH,
        )
        rx.start()

        rd.wait()
        rx.wait()
        out_ref[:, :] = xr_ref[:, :].astype(jnp.float32) + dr_ref[0, 0].astype(jnp.float32)

    return pl.pallas_call(
        body,
        out_shape=jax.ShapeDtypeStruct((m, n), jnp.float32),
        in_specs=[
            pl.BlockSpec(memory_space=pltpu.VMEM),
            pl.BlockSpec(memory_space=pltpu.VMEM),
        ],
        out_specs=pl.BlockSpec(memory_space=pltpu.VMEM),
        scratch_shapes=[
            pltpu.VMEM((m, n), jnp.bfloat16),
            pltpu.VMEM((m, n), jnp.bfloat16),
            pltpu.VMEM((1, m), jnp.int32),
            pltpu.SemaphoreType.DMA((4,)),
        ],
        compiler_params=pltpu.CompilerParams(collective_id=0),
    )(x, dest2)


# device time: 8507 ns/iter; 1.0695x vs baseline; 1.0695x over previous
import jax
import jax.numpy as jnp
from jax import lax
from jax.experimental import pallas as pl
from jax.experimental.pallas import tpu as pltpu

CH = 64


def kernel(x, dest):
    m, n = x.shape
    nch = m // CH
    dest2 = dest.reshape(1, m).astype(jnp.int32)

    def body(x_ref, d_ref, out_ref, xs_ref, st_ref, xr_ref, ssems, rsems):
        my_x = lax.axis_index("x")
        my_y = lax.axis_index("y")
        peer = (1 - my_x, my_y)
        is0 = my_x == 0
        i32 = jnp.int32
        f32 = jnp.float32
        bf16 = jnp.bfloat16

        barrier = pltpu.get_barrier_semaphore()
        pl.semaphore_signal(
            barrier, inc=1, device_id=peer, device_id_type=pl.DeviceIdType.MESH
        )

        xs_ref[:, :] = x_ref[:, :].astype(bf16)

        d = d_ref[:, :]
        sel_keep = d == my_x
        rows = lax.broadcasted_iota(i32, (m, m), 0)
        cols = lax.broadcasted_iota(i32, (m, m), 1)
        tri = (rows <= cols).astype(bf16)
        cum_keep = jnp.dot(
            sel_keep.astype(bf16), tri, preferred_element_type=f32
        ).astype(i32)
        c_keep = cum_keep[0, m - 1]
        q = m - c_keep

        idx1 = lax.broadcasted_iota(i32, (1, m), 1) + 1
        cum_send = idx1 - cum_keep

        dst_off = jnp.where(is0, 0, m - q)
        off_in = jnp.where(is0, m - q, 0)
        off_l = jnp.where(is0, 0, q)

        pos_send = dst_off + cum_send - 1
        g_send = ((rows == pos_send) & ~sel_keep).astype(bf16)
        st_ref[:, :] = jnp.dot(
            g_send, xs_ref[:, :], preferred_element_type=f32
        ).astype(bf16)

        pl.semaphore_wait(barrier, 1)

        descs = [
            pltpu.make_async_remote_copy(
                src_ref=st_ref.at[pl.ds(i * CH, CH)],
                dst_ref=xr_ref.at[pl.ds(i * CH, CH)],
                send_sem=ssems.at[i],
                recv_sem=rsems.at[i],
                device_id=peer,
                device_id_type=pl.DeviceIdType.MESH,
            )
            for i in range(nch)
        ]
        lo_s = dst_off // CH
        hi_s = (dst_off + q + CH - 1) // CH
        for i in range(nch):
            @pl.when((lo_s <= i) & (i < hi_s))
            def _():
                descs[i].start()

        pos_keep = off_l + cum_keep - 1
        g_keep = ((rows == pos_keep) & sel_keep).astype(bf16)
        partial = jnp.dot(g_keep, xs_ref[:, :], preferred_element_type=f32)

        lo_r = off_in // CH
        hi_r = (off_in + q + CH - 1) // CH
        for i in range(nch):
            @pl.when((lo_r <= i) & (i < hi_r))
            def _():
                descs[i].wait_recv()

        rcol = lax.broadcasted_iota(i32, (m, 1), 0)
        own = (rcol >= off_l) & (rcol < off_l + c_keep)
        out_ref[:, :] = jnp.where(own, partial, xr_ref[:, :].astype(f32))

        for i in range(nch):
            @pl.when((lo_s <= i) & (i < hi_s))
            def _():
                descs[i].wait_send()

    return pl.pallas_call(
        body,
        out_shape=jax.ShapeDtypeStruct((m, n), jnp.float32),
        in_specs=[
            pl.BlockSpec(memory_space=pltpu.VMEM),
            pl.BlockSpec(memory_space=pltpu.VMEM),
        ],
        out_specs=pl.BlockSpec(memory_space=pltpu.VMEM),
        scratch_shapes=[
            pltpu.VMEM((m, n), jnp.bfloat16),
            pltpu.VMEM((m, n), jnp.bfloat16),
            pltpu.VMEM((m, n), jnp.bfloat16),
            pltpu.SemaphoreType.DMA((m // CH,)),
            pltpu.SemaphoreType.DMA((m // CH,)),
        ],
        compiler_params=pltpu.CompilerParams(collective_id=0),
    )(x, dest2)


# device time: 8337 ns/iter; 1.0913x vs baseline; 1.0204x over previous
import jax
import jax.numpy as jnp
from jax import lax
from jax.experimental import pallas as pl
from jax.experimental.pallas import tpu as pltpu

CH = 64


def kernel(x, dest):
    m, n = x.shape
    nch = m // CH
    dest2 = dest.reshape(1, m).astype(jnp.int32)

    def body(x_ref, d_ref, out_ref, xs_ref, st_ref, xr_ref, ssems, rsems):
        my_x = lax.axis_index("x")
        my_y = lax.axis_index("y")
        peer = (1 - my_x, my_y)
        is0 = my_x == 0
        i32 = jnp.int32
        f32 = jnp.float32
        bf16 = jnp.bfloat16

        barrier = pltpu.get_barrier_semaphore()
        pl.semaphore_signal(
            barrier, inc=1, device_id=peer, device_id_type=pl.DeviceIdType.MESH
        )

        xs_ref[:, :] = x_ref[:, :].astype(bf16)

        d = d_ref[:, :]
        sel_keep = d == my_x
        rows = lax.broadcasted_iota(i32, (m, m), 0)
        cols = lax.broadcasted_iota(i32, (m, m), 1)
        tri = (rows <= cols).astype(bf16)
        cum_keep = jnp.dot(
            sel_keep.astype(bf16), tri, preferred_element_type=f32
        ).astype(i32)
        c_keep = cum_keep[0, m - 1]
        q = m - c_keep

        idx1 = lax.broadcasted_iota(i32, (1, m), 1) + 1
        cum_send = idx1 - cum_keep

        dst_off = jnp.where(is0, 0, m - q)
        off_in = jnp.where(is0, m - q, 0)
        off_l = jnp.where(is0, 0, q)

        pos_send = dst_off + cum_send - 1
        g_send = ((rows == pos_send) & ~sel_keep).astype(bf16)
        st_ref[:, :] = jnp.dot(
            g_send, xs_ref[:, :], preferred_element_type=f32
        ).astype(bf16)

        pl.semaphore_wait(barrier, 1)

        descs = [
            pltpu.make_async_remote_copy(
                src_ref=st_ref.at[pl.ds(u * CH, CH)],
                dst_ref=xr_ref.at[pl.ds(u * CH, CH)],
                send_sem=ssems.at[u],
                recv_sem=rsems.at[u],
                device_id=peer,
                device_id_type=pl.DeviceIdType.MESH,
            )
            for u in range(nch)
        ]
        lo_s = dst_off // CH
        hi_s = (dst_off + q + CH - 1) // CH
        for u in range(nch):
            @pl.when((lo_s <= u) & (u < hi_s))
            def _():
                descs[u].start()

        pos_keep = off_l + cum_keep - 1
        g_keep = ((rows == pos_keep) & sel_keep).astype(bf16)
        partial = jnp.dot(
            g_keep, xs_ref[:, :], preferred_element_type=f32
        ).astype(bf16)

        rcol = lax.broadcasted_iota(i32, (m, 1), 0)
        own = (rcol >= off_l) & (rcol < off_l + c_keep)
        lo_r = off_in // CH
        hi_r = (off_in + q + CH - 1) // CH
        for u in range(nch):
            @pl.when((lo_r <= u) & (u < hi_r))
            def _():
                descs[u].wait_recv()

            a, b = u * CH, (u + 1) * CH
            out_ref[a:b, :] = jnp.where(
                own[a:b], partial[a:b, :], xr_ref[a:b, :]
            )

        for u in range(nch):
            @pl.when((lo_s <= u) & (u < hi_s))
            def _():
                descs[u].wait_send()

    return pl.pallas_call(
        body,
        out_shape=jax.ShapeDtypeStruct((m, n), jnp.bfloat16),
        in_specs=[
            pl.BlockSpec(memory_space=pltpu.VMEM),
            pl.BlockSpec(memory_space=pltpu.VMEM),
        ],
        out_specs=pl.BlockSpec(memory_space=pltpu.VMEM),
        scratch_shapes=[
            pltpu.VMEM((m, n), jnp.bfloat16),
            pltpu.VMEM((m, n), jnp.bfloat16),
            pltpu.VMEM((m, n), jnp.bfloat16),
            pltpu.SemaphoreType.DMA((m // CH,)),
            pltpu.SemaphoreType.DMA((m // CH,)),
        ],
        compiler_params=pltpu.CompilerParams(collective_id=0),
    )(x, dest2)


# device time: 8335 ns/iter; 1.0915x vs baseline; 1.0002x over previous
import jax
import jax.numpy as jnp
from jax import lax
from jax.experimental import pallas as pl
from jax.experimental.pallas import tpu as pltpu

CH = 64


def kernel(x, dest):
    m, n = x.shape
    nch = m // CH
    dest2 = dest.reshape(1, m).astype(jnp.int32)

    def body(x_ref, d_ref, out_ref, xs_ref, st_ref, xr_ref, ssems, rsems):
        my_x = lax.axis_index("x")
        my_y = lax.axis_index("y")
        peer = (1 - my_x, my_y)
        is0 = my_x == 0
        i32 = jnp.int32
        f32 = jnp.float32
        bf16 = jnp.bfloat16

        barrier = pltpu.get_barrier_semaphore()
        pl.semaphore_signal(
            barrier, inc=1, device_id=peer, device_id_type=pl.DeviceIdType.MESH
        )

        xs_ref[:, :] = x_ref[:, :].astype(bf16)

        d = d_ref[:, :]
        sel_keep = d == my_x
        rows = lax.broadcasted_iota(i32, (m, m), 0)
        cols = lax.broadcasted_iota(i32, (m, m), 1)
        tri = (rows <= cols).astype(bf16)
        cum_keep = jnp.dot(
            sel_keep.astype(bf16), tri, preferred_element_type=f32
        ).astype(i32)
        c_keep = cum_keep[0, m - 1]
        q = m - c_keep

        idx1 = lax.broadcasted_iota(i32, (1, m), 1) + 1
        cum_send = idx1 - cum_keep

        dst_off = jnp.where(is0, 0, m - q)
        off_in = jnp.where(is0, m - q, 0)
        off_l = jnp.where(is0, 0, q)

        pos_send = dst_off + cum_send - 1
        g_send = ((rows == pos_send) & ~sel_keep).astype(bf16)
        st_ref[:, :] = jnp.dot(
            g_send, xs_ref[:, :], preferred_element_type=f32
        ).astype(bf16)

        pl.semaphore_wait(barrier, 1)

        descs = [
            pltpu.make_async_remote_copy(
                src_ref=st_ref.at[pl.ds(u * CH, CH)],
                dst_ref=xr_ref.at[pl.ds(u * CH, CH)],
                send_sem=ssems.at[u],
                recv_sem=rsems.at[u],
                device_id=peer,
                device_id_type=pl.DeviceIdType.MESH,
            )
            for u in range(nch)
        ]
        lo_s = dst_off // CH
        hi_s = (dst_off + q + CH - 1) // CH
        for u in range(nch):
            @pl.when((lo_s <= u) & (u < hi_s))
            def _():
                descs[u].start()

        pos_keep = off_l + cum_keep - 1
        g_keep = ((rows == pos_keep) & sel_keep).astype(bf16)
        partial = jnp.dot(
            g_keep, xs_ref[:, :], preferred_element_type=f32
        ).astype(bf16)

        rcol = lax.broadcasted_iota(i32, (m, 1), 0)
        own = (rcol >= off_l) & (rcol < off_l + c_keep)
        lo_r = off_in // CH
        hi_r = (off_in + q + CH - 1) // CH
        for u in range(nch):
            a, b = u * CH, (u + 1) * CH
            in_range = (lo_r <= u) & (u < hi_r)

            @pl.when(~in_range)
            def _():
                out_ref[a:b, :] = partial[a:b, :]

        for u in range(nch):
            a, b = u * CH, (u + 1) * CH
            in_range = (lo_r <= u) & (u < hi_r)

            @pl.when(in_range)
            def _():
                descs[u].wait_recv()
                out_ref[a:b, :] = jnp.where(
                    own[a:b], partial[a:b, :], xr_ref[a:b, :]
                )

        for u in range(nch):
            @pl.when((lo_s <= u) & (u < hi_s))
            def _():
                descs[u].wait_send()

    return pl.pallas_call(
        body,
        out_shape=jax.ShapeDtypeStruct((m, n), jnp.bfloat16),
        in_specs=[
            pl.BlockSpec(memory_space=pltpu.VMEM),
            pl.BlockSpec(memory_space=pltpu.VMEM),
        ],
        out_specs=pl.BlockSpec(memory_space=pltpu.VMEM),
        scratch_shapes=[
            pltpu.VMEM((m, n), jnp.bfloat16),
            pltpu.VMEM((m, n), jnp.bfloat16),
            pltpu.VMEM((m, n), jnp.bfloat16),
            pltpu.SemaphoreType.DMA((m // CH,)),
            pltpu.SemaphoreType.DMA((m // CH,)),
        ],
        compiler_params=pltpu.CompilerParams(collective_id=0),
    )(x, dest2)
